# baseline (speedup 1.0000x reference)
import jax
import jax.numpy as jnp
from jax import lax
from jax.experimental import pallas as pl
from jax.experimental.pallas import tpu as pltpu
from jax.experimental.pallas import tpu_sc as plsc

N = 10000
EDGES = 320000
D = 128
NP = 10240
NC = 2
NS = 16
NW = NC * NS
EPT = 10240
CHUNK = 128
NCHUNK = EPT // CHUNK
EPAD = EPT * NW
RPT = NP // NS
BLK = 256
DUMMY = N

_mesh = plsc.VectorSubcoreMesh(
    core_axis_name="c", subcore_axis_name="s", num_cores=NC, num_subcores=NS
)


def _tile_ids():
    cid = lax.axis_index("c")
    sid = lax.axis_index("s")
    return cid, sid, cid * NS + sid


def _deg_body(dst_hbm, out0_hbm, out1_hbm, dstv, ones_v, zbuf, accd,
              ssem0, ssem1):
    cid, sid, wid = _tile_ids()
    for i in range(RPT // 16):
        zbuf[pl.ds(i * 16, 16)] = jnp.zeros((16,), jnp.float32)
    for i in range(CHUNK // 16):
        ones_v[pl.ds(i * 16, 16)] = jnp.ones((16,), jnp.float32)
    row0 = pl.multiple_of(sid * RPT, 8)
    pltpu.sync_copy(zbuf, accd.at[pl.ds(row0, RPT)])
    pltpu.sync_copy(dst_hbm.at[wid], dstv)
    plsc.subcore_barrier()

    sems = (ssem0, ssem1)

    def fire(j, b):
        pltpu.async_copy(ones_v, accd.at[dstv.at[j]], sems[b], add=True)

    def drain(j, b):
        pltpu.make_async_copy(ones_v, accd.at[dstv.at[j]], sems[b]).wait()

    fire(0, 0)
    fire(1, 1)

    def body(i, carry):
        j0 = 2 + 2 * i
        drain(j0, 0)
        fire(j0, 0)
        drain(j0 + 1, 1)
        fire(j0 + 1, 1)
        return carry

    lax.fori_loop(0, (NCHUNK - 2) // 2, body, 0)
    drain(0, 0)
    drain(1, 1)
    plsc.subcore_barrier()

    @pl.when(cid == 0)
    def _():
        pltpu.sync_copy(accd.at[pl.ds(row0, RPT)], out0_hbm.at[pl.ds(row0, RPT)])

    @pl.when(cid == 1)
    def _():
        pltpu.sync_copy(accd.at[pl.ds(row0, RPT)], out1_hbm.at[pl.ds(row0, RPT)])


_deg_call = pl.kernel(
    _deg_body,
    out_type=(
        jax.ShapeDtypeStruct((NP,), jnp.float32),
        jax.ShapeDtypeStruct((NP,), jnp.float32),
    ),
    mesh=_mesh,
    scratch_types=[
        pltpu.VMEM((NCHUNK, CHUNK), jnp.int32),
        pltpu.VMEM((CHUNK,), jnp.float32),
        pltpu.VMEM((RPT,), jnp.float32),
        pltpu.VMEM_SHARED((NP,), jnp.float32),
        pltpu.SemaphoreType.DMA,
        pltpu.SemaphoreType.DMA,
    ],
)


SEG = 40
NSEG = NCHUNK // SEG


def _scat_body(g_hbm, src_hbm, dst_hbm, out0_hbm, out1_hbm, srcv, dstv,
               rows, acc, gsem, ssem0, ssem1):
    cid, sid, wid = _tile_ids()
    row0 = pl.multiple_of(sid * RPT, 8)
    pltpu.sync_copy(g_hbm.at[pl.ds(row0, RPT)], acc.at[pl.ds(row0, RPT)])
    plsc.subcore_barrier()

    sems = (ssem0, ssem1)

    def gather(j, b):
        h = CHUNK // 2
        d0 = pltpu.async_copy(
            g_hbm.at[srcv.at[j, pl.ds(0, h)]], rows.at[b, pl.ds(0, h)], gsem)
        d1 = pltpu.async_copy(
            g_hbm.at[srcv.at[j, pl.ds(h, h)]], rows.at[b, pl.ds(h, h)], gsem)
        d0.wait()
        d1.wait()

    def fire_scatter(j, b):
        pltpu.async_copy(rows.at[b], acc.at[dstv.at[j]], sems[b], add=True)

    def drain_scatter(j, b):
        pltpu.make_async_copy(rows.at[b], acc.at[dstv.at[j]], sems[b]).wait()

    for seg in range(NSEG):
        if seg > 0:
            drain_scatter(0, 0)
            drain_scatter(1, 1)
        pltpu.sync_copy(src_hbm.at[wid, pl.ds(seg * SEG, SEG)], srcv)
        pltpu.sync_copy(dst_hbm.at[wid, pl.ds(seg * SEG, SEG)], dstv)
        gather(0, 0)
        fire_scatter(0, 0)
        gather(1, 1)
        fire_scatter(1, 1)

        def body(i, carry):
            j0 = 2 + 2 * i
            drain_scatter(j0, 0)
            gather(j0, 0)
            fire_scatter(j0, 0)
            j1 = j0 + 1
            drain_scatter(j1, 1)
            gather(j1, 1)
            fire_scatter(j1, 1)
            return carry

        lax.fori_loop(0, (SEG - 2) // 2, body, 0)
    drain_scatter(0, 0)
    drain_scatter(1, 1)
    plsc.subcore_barrier()

    @pl.when(cid == 0)
    def _():
        pltpu.sync_copy(acc.at[pl.ds(row0, RPT)], out0_hbm.at[pl.ds(row0, RPT)])

    @pl.when(cid == 1)
    def _():
        pltpu.sync_copy(acc.at[pl.ds(row0, RPT)], out1_hbm.at[pl.ds(row0, RPT)])


_scat_call = pl.kernel(
    _scat_body,
    out_type=(
        jax.ShapeDtypeStruct((NP, D), jnp.float32),
        jax.ShapeDtypeStruct((NP, D), jnp.float32),
    ),
    mesh=_mesh,
    scratch_types=[
        pltpu.VMEM((SEG, CHUNK), jnp.int32),
        pltpu.VMEM((SEG, CHUNK), jnp.int32),
        pltpu.VMEM((2, CHUNK, D), jnp.float32),
        pltpu.VMEM_SHARED((NP, D), jnp.float32),
        pltpu.SemaphoreType.DMA,
        pltpu.SemaphoreType.DMA,
        pltpu.SemaphoreType.DMA,
    ],
)


def _dcol(k0_ref, k1_ref):
    return lax.rsqrt(k0_ref[...] + k1_ref[...] + 1.0)


def _tc1_body(x_ref, w_ref, k0_ref, k1_ref, o_ref):
    o_ref[...] = (
        jnp.dot(x_ref[...], w_ref[...], preferred_element_type=jnp.float32)
        * _dcol(k0_ref, k1_ref)
    )


def _tc1(x_p, W1, k0, k1):
    return pl.pallas_call(
        _tc1_body,
        grid=(NP // BLK,),
        in_specs=[
            pl.BlockSpec((BLK, D), lambda i: (i, 0)),
            pl.BlockSpec((D, D), lambda i: (0, 0)),
            pl.BlockSpec((BLK, 1), lambda i: (i, 0)),
            pl.BlockSpec((BLK, 1), lambda i: (i, 0)),
        ],
        out_specs=pl.BlockSpec((BLK, D), lambda i: (i, 0)),
        out_shape=jax.ShapeDtypeStruct((NP, D), jnp.float32),
    )(x_p, W1, k0, k1)


def _tc2_body(p0_ref, p1_ref, g1_ref, k0_ref, k1_ref, b1_ref, w2_ref, o_ref):
    d = _dcol(k0_ref, k1_ref)
    acc = p0_ref[...] + p1_ref[...] - g1_ref[...]
    h1 = jnp.maximum(d * acc + b1_ref[...], 0.0)
    o_ref[...] = (
        jnp.dot(h1, w2_ref[...], preferred_element_type=jnp.float32) * d
    )


def _tc2(p0, p1, g1, k0, k1, b1, W2):
    return pl.pallas_call(
        _tc2_body,
        grid=(NP // BLK,),
        in_specs=[
            pl.BlockSpec((BLK, D), lambda i: (i, 0)),
            pl.BlockSpec((BLK, D), lambda i: (i, 0)),
            pl.BlockSpec((BLK, D), lambda i: (i, 0)),
            pl.BlockSpec((BLK, 1), lambda i: (i, 0)),
            pl.BlockSpec((BLK, 1), lambda i: (i, 0)),
            pl.BlockSpec((1, D), lambda i: (0, 0)),
            pl.BlockSpec((D, D), lambda i: (0, 0)),
        ],
        out_specs=pl.BlockSpec((BLK, D), lambda i: (i, 0)),
        out_shape=jax.ShapeDtypeStruct((NP, D), jnp.float32),
    )(p0, p1, g1, k0, k1, b1, W2)


def _tc3_body(q0_ref, q1_ref, g2_ref, k0_ref, k1_ref, b2_ref, wm1_ref,
              bm1_ref, wm2_ref, bm2_ref, o_ref):
    d = _dcol(k0_ref, k1_ref)
    h2 = d * (q0_ref[...] + q1_ref[...] - g2_ref[...]) + b2_ref[...]
    h3 = jnp.maximum(
        jnp.dot(h2, wm1_ref[...], preferred_element_type=jnp.float32)
        + bm1_ref[...],
        0.0,
    )
    o_ref[...] = (
        jnp.dot(h3, wm2_ref[...], preferred_element_type=jnp.float32)
        + bm2_ref[...]
    )


def _tc3(q0, q1, g2, k0, k1, b2, Wm1, bm1, Wm2, bm2):
    return pl.pallas_call(
        _tc3_body,
        grid=(NP // BLK,),
        in_specs=[
            pl.BlockSpec((BLK, D), lambda i: (i, 0)),
            pl.BlockSpec((BLK, D), lambda i: (i, 0)),
            pl.BlockSpec((BLK, D), lambda i: (i, 0)),
            pl.BlockSpec((BLK, 1), lambda i: (i, 0)),
            pl.BlockSpec((BLK, 1), lambda i: (i, 0)),
            pl.BlockSpec((1, D), lambda i: (0, 0)),
            pl.BlockSpec((D, D), lambda i: (0, 0)),
            pl.BlockSpec((1, D), lambda i: (0, 0)),
            pl.BlockSpec((D, D), lambda i: (0, 0)),
            pl.BlockSpec((1, D), lambda i: (0, 0)),
        ],
        out_specs=pl.BlockSpec((BLK, D), lambda i: (i, 0)),
        out_shape=jax.ShapeDtypeStruct((NP, D), jnp.float32),
    )(q0, q1, g2, k0, k1, b2, Wm1, bm1, Wm2, bm2)


def kernel(x, edge_index, W1, b1, W2, b2, Wm1, bm1, Wm2, bm2):
    src = edge_index[0]
    dst = edge_index[1]
    pad = EPAD - EDGES
    pad_i = jnp.arange(pad, dtype=jnp.int32)
    src_p = jnp.concatenate([src, pad_i % N])
    dst_p = jnp.concatenate([dst, DUMMY + pad_i % (NP - N)])
    src_p = src_p.reshape(NW, NCHUNK, CHUNK)
    dst_p = dst_p.reshape(NW, NCHUNK, CHUNK)
    x_p = jnp.concatenate([x, jnp.zeros((NP - N, D), x.dtype)])

    deg0, deg1 = _deg_call(dst_p)
    k0 = deg0.reshape(NP, 1)
    k1 = deg1.reshape(NP, 1)
    g1 = _tc1(x_p, W1, k0, k1)
    p0, p1 = _scat_call(g1, src_p, dst_p)
    g2 = _tc2(p0, p1, g1, k0, k1, b1.reshape(1, D), W2)
    q0, q1 = _scat_call(g2, src_p, dst_p)
    out = _tc3(q0, q1, g2, k0, k1, b2.reshape(1, D), Wm1,
               bm1.reshape(1, D), Wm2, bm2.reshape(1, D))
    return out[:N]


# speedup vs baseline: 1.1187x; 1.1187x over previous
import jax
import jax.numpy as jnp
from jax import lax
from jax.experimental import pallas as pl
from jax.experimental.pallas import tpu as pltpu
from jax.experimental.pallas import tpu_sc as plsc

N = 10000
EDGES = 320000
D = 128
NP = 10240
NC = 2
NS = 16
NW = NC * NS
EPT = 10240
CHUNK = 128
NCHUNK = EPT // CHUNK
EPAD = EPT * NW
RPT = NP // NS
BLK = 256
DUMMY = N

_mesh = plsc.VectorSubcoreMesh(
    core_axis_name="c", subcore_axis_name="s", num_cores=NC, num_subcores=NS
)


def _tile_ids():
    cid = lax.axis_index("c")
    sid = lax.axis_index("s")
    return cid, sid, cid * NS + sid


def _deg_body(dst_hbm, out0_hbm, out1_hbm, dstv, ones_v, zbuf, accd,
              ssem0, ssem1):
    cid, sid, wid = _tile_ids()
    for i in range(RPT // 16):
        zbuf[pl.ds(i * 16, 16)] = jnp.zeros((16,), jnp.float32)
    for i in range(CHUNK // 16):
        ones_v[pl.ds(i * 16, 16)] = jnp.ones((16,), jnp.float32)
    row0 = pl.multiple_of(sid * RPT, 8)
    pltpu.sync_copy(zbuf, accd.at[pl.ds(row0, RPT)])
    pltpu.sync_copy(dst_hbm.at[wid], dstv)
    plsc.subcore_barrier()

    sems = (ssem0, ssem1)

    def fire(j, b):
        pltpu.async_copy(ones_v, accd.at[dstv.at[j]], sems[b], add=True)

    def drain(j, b):
        pltpu.make_async_copy(ones_v, accd.at[dstv.at[j]], sems[b]).wait()

    fire(0, 0)
    fire(1, 1)

    def body(i, carry):
        j0 = 2 + 2 * i
        drain(j0, 0)
        fire(j0, 0)
        drain(j0 + 1, 1)
        fire(j0 + 1, 1)
        return carry

    lax.fori_loop(0, (NCHUNK - 2) // 2, body, 0)
    drain(0, 0)
    drain(1, 1)
    plsc.subcore_barrier()

    @pl.when(cid == 0)
    def _():
        pltpu.sync_copy(accd.at[pl.ds(row0, RPT)], out0_hbm.at[pl.ds(row0, RPT)])

    @pl.when(cid == 1)
    def _():
        pltpu.sync_copy(accd.at[pl.ds(row0, RPT)], out1_hbm.at[pl.ds(row0, RPT)])


_deg_call = pl.kernel(
    _deg_body,
    out_type=(
        jax.ShapeDtypeStruct((NP,), jnp.float32),
        jax.ShapeDtypeStruct((NP,), jnp.float32),
    ),
    mesh=_mesh,
    scratch_types=[
        pltpu.VMEM((NCHUNK, CHUNK), jnp.int32),
        pltpu.VMEM((CHUNK,), jnp.float32),
        pltpu.VMEM((RPT,), jnp.float32),
        pltpu.VMEM_SHARED((NP,), jnp.float32),
        pltpu.SemaphoreType.DMA,
        pltpu.SemaphoreType.DMA,
    ],
)


SEG = 40
NSEG = NCHUNK // SEG


def _scat_body(g_hbm, src_hbm, dst_hbm, out0_hbm, out1_hbm, srcv, dstv,
               rows, acc, gsem0, gsem1, ssem0, ssem1):
    cid, sid, wid = _tile_ids()
    row0 = pl.multiple_of(sid * RPT, 8)
    pltpu.sync_copy(g_hbm.at[pl.ds(row0, RPT)], acc.at[pl.ds(row0, RPT)])
    plsc.subcore_barrier()

    gsems = (gsem0, gsem1)
    sems = (ssem0, ssem1)

    def fire_gather(j, b):
        pltpu.async_copy(g_hbm.at[srcv.at[j]], rows.at[b], gsems[b])

    def wait_gather(j, b):
        pltpu.make_async_copy(g_hbm.at[srcv.at[j]], rows.at[b], gsems[b]).wait()

    def fire_scatter(j, b):
        pltpu.async_copy(rows.at[b], acc.at[dstv.at[j]], sems[b], add=True)

    def drain_scatter(j, b):
        pltpu.make_async_copy(rows.at[b], acc.at[dstv.at[j]], sems[b]).wait()

    for seg in range(NSEG):
        pltpu.sync_copy(src_hbm.at[wid, pl.ds(seg * SEG, SEG)], srcv)
        pltpu.sync_copy(dst_hbm.at[wid, pl.ds(seg * SEG, SEG)], dstv)
        fire_gather(0, 0)
        fire_gather(1, 1)
        wait_gather(0, 0)
        fire_scatter(0, 0)

        def body(i, carry):
            j0 = 2 + 2 * i
            drain_scatter(j0 - 2, 0)
            fire_gather(j0, 0)
            wait_gather(j0 - 1, 1)
            fire_scatter(j0 - 1, 1)
            j1 = j0 + 1
            drain_scatter(j1 - 2, 1)
            fire_gather(j1, 1)
            wait_gather(j1 - 1, 0)
            fire_scatter(j1 - 1, 0)
            return carry

        lax.fori_loop(0, (SEG - 2) // 2, body, 0)
        wait_gather(SEG - 1, 1)
        fire_scatter(SEG - 1, 1)
        drain_scatter(SEG - 2, 0)
        drain_scatter(SEG - 1, 1)
    plsc.subcore_barrier()

    @pl.when(cid == 0)
    def _():
        pltpu.sync_copy(acc.at[pl.ds(row0, RPT)], out0_hbm.at[pl.ds(row0, RPT)])

    @pl.when(cid == 1)
    def _():
        pltpu.sync_copy(acc.at[pl.ds(row0, RPT)], out1_hbm.at[pl.ds(row0, RPT)])


_scat_call = pl.kernel(
    _scat_body,
    out_type=(
        jax.ShapeDtypeStruct((NP, D), jnp.float32),
        jax.ShapeDtypeStruct((NP, D), jnp.float32),
    ),
    mesh=_mesh,
    scratch_types=[
        pltpu.VMEM((SEG, CHUNK), jnp.int32),
        pltpu.VMEM((SEG, CHUNK), jnp.int32),
        pltpu.VMEM((2, CHUNK, D), jnp.float32),
        pltpu.VMEM_SHARED((NP, D), jnp.float32),
        pltpu.SemaphoreType.DMA,
        pltpu.SemaphoreType.DMA,
        pltpu.SemaphoreType.DMA,
        pltpu.SemaphoreType.DMA,
    ],
)


def _dcol(k0_ref, k1_ref):
    return lax.rsqrt(k0_ref[...] + k1_ref[...] + 1.0)


def _tc1_body(x_ref, w_ref, k0_ref, k1_ref, o_ref):
    o_ref[...] = (
        jnp.dot(x_ref[...], w_ref[...], preferred_element_type=jnp.float32)
        * _dcol(k0_ref, k1_ref)
    )


def _tc1(x_p, W1, k0, k1):
    return pl.pallas_call(
        _tc1_body,
        grid=(NP // BLK,),
        in_specs=[
            pl.BlockSpec((BLK, D), lambda i: (i, 0)),
            pl.BlockSpec((D, D), lambda i: (0, 0)),
            pl.BlockSpec((BLK, 1), lambda i: (i, 0)),
            pl.BlockSpec((BLK, 1), lambda i: (i, 0)),
        ],
        out_specs=pl.BlockSpec((BLK, D), lambda i: (i, 0)),
        out_shape=jax.ShapeDtypeStruct((NP, D), jnp.float32),
    )(x_p, W1, k0, k1)


def _tc2_body(p0_ref, p1_ref, g1_ref, k0_ref, k1_ref, b1_ref, w2_ref, o_ref):
    d = _dcol(k0_ref, k1_ref)
    acc = p0_ref[...] + p1_ref[...] - g1_ref[...]
    h1 = jnp.maximum(d * acc + b1_ref[...], 0.0)
    o_ref[...] = (
        jnp.dot(h1, w2_ref[...], preferred_element_type=jnp.float32) * d
    )


def _tc2(p0, p1, g1, k0, k1, b1, W2):
    return pl.pallas_call(
        _tc2_body,
        grid=(NP // BLK,),
        in_specs=[
            pl.BlockSpec((BLK, D), lambda i: (i, 0)),
            pl.BlockSpec((BLK, D), lambda i: (i, 0)),
            pl.BlockSpec((BLK, D), lambda i: (i, 0)),
            pl.BlockSpec((BLK, 1), lambda i: (i, 0)),
            pl.BlockSpec((BLK, 1), lambda i: (i, 0)),
            pl.BlockSpec((1, D), lambda i: (0, 0)),
            pl.BlockSpec((D, D), lambda i: (0, 0)),
        ],
        out_specs=pl.BlockSpec((BLK, D), lambda i: (i, 0)),
        out_shape=jax.ShapeDtypeStruct((NP, D), jnp.float32),
    )(p0, p1, g1, k0, k1, b1, W2)


def _tc3_body(q0_ref, q1_ref, g2_ref, k0_ref, k1_ref, b2_ref, wm1_ref,
              bm1_ref, wm2_ref, bm2_ref, o_ref):
    d = _dcol(k0_ref, k1_ref)
    h2 = d * (q0_ref[...] + q1_ref[...] - g2_ref[...]) + b2_ref[...]
    h3 = jnp.maximum(
        jnp.dot(h2, wm1_ref[...], preferred_element_type=jnp.float32)
        + bm1_ref[...],
        0.0,
    )
    o_ref[...] = (
        jnp.dot(h3, wm2_ref[...], preferred_element_type=jnp.float32)
        + bm2_ref[...]
    )


def _tc3(q0, q1, g2, k0, k1, b2, Wm1, bm1, Wm2, bm2):
    return pl.pallas_call(
        _tc3_body,
        grid=(NP // BLK,),
        in_specs=[
            pl.BlockSpec((BLK, D), lambda i: (i, 0)),
            pl.BlockSpec((BLK, D), lambda i: (i, 0)),
            pl.BlockSpec((BLK, D), lambda i: (i, 0)),
            pl.BlockSpec((BLK, 1), lambda i: (i, 0)),
            pl.BlockSpec((BLK, 1), lambda i: (i, 0)),
            pl.BlockSpec((1, D), lambda i: (0, 0)),
            pl.BlockSpec((D, D), lambda i: (0, 0)),
            pl.BlockSpec((1, D), lambda i: (0, 0)),
            pl.BlockSpec((D, D), lambda i: (0, 0)),
            pl.BlockSpec((1, D), lambda i: (0, 0)),
        ],
        out_specs=pl.BlockSpec((BLK, D), lambda i: (i, 0)),
        out_shape=jax.ShapeDtypeStruct((NP, D), jnp.float32),
    )(q0, q1, g2, k0, k1, b2, Wm1, bm1, Wm2, bm2)


def kernel(x, edge_index, W1, b1, W2, b2, Wm1, bm1, Wm2, bm2):
    src = edge_index[0]
    dst = edge_index[1]
    pad = EPAD - EDGES
    pad_i = jnp.arange(pad, dtype=jnp.int32)
    src_p = jnp.concatenate([src, pad_i % N])
    dst_p = jnp.concatenate([dst, DUMMY + pad_i % (NP - N)])
    src_p = src_p.reshape(NW, NCHUNK, CHUNK)
    dst_p = dst_p.reshape(NW, NCHUNK, CHUNK)
    x_p = jnp.concatenate([x, jnp.zeros((NP - N, D), x.dtype)])

    deg0, deg1 = _deg_call(dst_p)
    k0 = deg0.reshape(NP, 1)
    k1 = deg1.reshape(NP, 1)
    g1 = _tc1(x_p, W1, k0, k1)
    p0, p1 = _scat_call(g1, src_p, dst_p)
    g2 = _tc2(p0, p1, g1, k0, k1, b1.reshape(1, D), W2)
    q0, q1 = _scat_call(g2, src_p, dst_p)
    out = _tc3(q0, q1, g2, k0, k1, b2.reshape(1, D), Wm1,
               bm1.reshape(1, D), Wm2, bm2.reshape(1, D))
    return out[:N]


# speedup vs baseline: 1.1304x; 1.0105x over previous
import jax
import jax.numpy as jnp
from jax import lax
from jax.experimental import pallas as pl
from jax.experimental.pallas import tpu as pltpu
from jax.experimental.pallas import tpu_sc as plsc

N = 10000
EDGES = 320000
D = 128
NP = 10240
NC = 2
NS = 16
NW = NC * NS
CHUNK = 96
NCHUNK = 108
EPT = NCHUNK * CHUNK
EPAD = EPT * NW
SEG = 36
NSEG = NCHUNK // SEG
NBUF = 3
RPT = NP // NS
BLK = 256
DUMMY = N

_mesh = plsc.VectorSubcoreMesh(
    core_axis_name="c", subcore_axis_name="s", num_cores=NC, num_subcores=NS
)


def _tile_ids():
    cid = lax.axis_index("c")
    sid = lax.axis_index("s")
    return cid, sid, cid * NS + sid


def _deg_body(dst_hbm, out0_hbm, out1_hbm, dstv, ones_v, zbuf, accd,
              ssem0, ssem1):
    cid, sid, wid = _tile_ids()
    for i in range(RPT // 16):
        zbuf[pl.ds(i * 16, 16)] = jnp.zeros((16,), jnp.float32)
    for i in range(CHUNK // 16):
        ones_v[pl.ds(i * 16, 16)] = jnp.ones((16,), jnp.float32)
    row0 = pl.multiple_of(sid * RPT, 8)
    pltpu.sync_copy(zbuf, accd.at[pl.ds(row0, RPT)])
    plsc.subcore_barrier()

    sems = (ssem0, ssem1)

    def fire(j, b):
        pltpu.async_copy(ones_v, accd.at[dstv.at[j]], sems[b], add=True)

    def drain(j, b):
        pltpu.make_async_copy(ones_v, accd.at[dstv.at[j]], sems[b]).wait()

    for seg in range(NSEG):
        if seg > 0:
            drain(0, 0)
            drain(1, 1)
        pltpu.sync_copy(dst_hbm.at[wid, seg], dstv)
        fire(0, 0)
        fire(1, 1)

        def body(i, carry):
            j0 = 2 + 2 * i
            drain(j0, 0)
            fire(j0, 0)
            drain(j0 + 1, 1)
            fire(j0 + 1, 1)
            return carry

        lax.fori_loop(0, (SEG - 2) // 2, body, 0)
    drain(0, 0)
    drain(1, 1)
    plsc.subcore_barrier()

    @pl.when(cid == 0)
    def _():
        pltpu.sync_copy(accd.at[pl.ds(row0, RPT)], out0_hbm.at[pl.ds(row0, RPT)])

    @pl.when(cid == 1)
    def _():
        pltpu.sync_copy(accd.at[pl.ds(row0, RPT)], out1_hbm.at[pl.ds(row0, RPT)])


_deg_call = pl.kernel(
    _deg_body,
    out_type=(
        jax.ShapeDtypeStruct((NP,), jnp.float32),
        jax.ShapeDtypeStruct((NP,), jnp.float32),
    ),
    mesh=_mesh,
    scratch_types=[
        pltpu.VMEM((SEG, CHUNK), jnp.int32),
        pltpu.VMEM((CHUNK,), jnp.float32),
        pltpu.VMEM((RPT,), jnp.float32),
        pltpu.VMEM_SHARED((NP,), jnp.float32),
        pltpu.SemaphoreType.DMA,
        pltpu.SemaphoreType.DMA,
    ],
)


def _scat_body(g_hbm, src_hbm, dst_hbm, out0_hbm, out1_hbm, srcv, dstv,
               rows, acc, gsem0, gsem1, gsem2, ssem0, ssem1, ssem2):
    cid, sid, wid = _tile_ids()
    row0 = pl.multiple_of(sid * RPT, 8)
    pltpu.sync_copy(g_hbm.at[pl.ds(row0, RPT)], acc.at[pl.ds(row0, RPT)])
    plsc.subcore_barrier()

    gsems = (gsem0, gsem1, gsem2)
    sems = (ssem0, ssem1, ssem2)

    def fire_gather(j, b):
        pltpu.async_copy(g_hbm.at[srcv.at[j]], rows.at[b], gsems[b])

    def wait_gather(j, b):
        pltpu.make_async_copy(g_hbm.at[srcv.at[j]], rows.at[b], gsems[b]).wait()

    def fire_scatter(j, b):
        pltpu.async_copy(rows.at[b], acc.at[dstv.at[j]], sems[b], add=True)

    def drain_scatter(j, b):
        pltpu.make_async_copy(rows.at[b], acc.at[dstv.at[j]], sems[b]).wait()

    for seg in range(NSEG):
        pltpu.sync_copy(src_hbm.at[wid, seg], srcv)
        pltpu.sync_copy(dst_hbm.at[wid, seg], dstv)
        fire_gather(0, 0)
        fire_gather(1, 1)
        wait_gather(0, 0)
        fire_scatter(0, 0)
        fire_gather(2, 2)
        wait_gather(1, 1)
        fire_scatter(1, 1)

        def body(m, carry):
            j0 = 3 + 3 * m
            for t in range(NBUF):
                j = j0 + t
                drain_scatter(j - 3, t)
                fire_gather(j, t)
                wait_gather(j - 1, (t + 2) % NBUF)
                fire_scatter(j - 1, (t + 2) % NBUF)
            return carry

        lax.fori_loop(0, (SEG - 3) // 3, body, 0)
        wait_gather(SEG - 1, 2)
        fire_scatter(SEG - 1, 2)
        drain_scatter(SEG - 3, 0)
        drain_scatter(SEG - 2, 1)
        drain_scatter(SEG - 1, 2)
    plsc.subcore_barrier()

    @pl.when(cid == 0)
    def _():
        pltpu.sync_copy(acc.at[pl.ds(row0, RPT)], out0_hbm.at[pl.ds(row0, RPT)])

    @pl.when(cid == 1)
    def _():
        pltpu.sync_copy(acc.at[pl.ds(row0, RPT)], out1_hbm.at[pl.ds(row0, RPT)])


_scat_call = pl.kernel(
    _scat_body,
    out_type=(
        jax.ShapeDtypeStruct((NP, D), jnp.float32),
        jax.ShapeDtypeStruct((NP, D), jnp.float32),
    ),
    mesh=_mesh,
    scratch_types=[
        pltpu.VMEM((SEG, CHUNK), jnp.int32),
        pltpu.VMEM((SEG, CHUNK), jnp.int32),
        pltpu.VMEM((NBUF, CHUNK, D), jnp.float32),
        pltpu.VMEM_SHARED((NP, D), jnp.float32),
        pltpu.SemaphoreType.DMA,
        pltpu.SemaphoreType.DMA,
        pltpu.SemaphoreType.DMA,
        pltpu.SemaphoreType.DMA,
        pltpu.SemaphoreType.DMA,
        pltpu.SemaphoreType.DMA,
    ],
)


def _dcol(k0_ref, k1_ref):
    return lax.rsqrt(k0_ref[...] + k1_ref[...] + 1.0)


def _tc1_body(x_ref, w_ref, k0_ref, k1_ref, o_ref):
    o_ref[...] = (
        jnp.dot(x_ref[...], w_ref[...], preferred_element_type=jnp.float32)
        * _dcol(k0_ref, k1_ref)
    )


def _tc1(x_p, W1, k0, k1):
    return pl.pallas_call(
        _tc1_body,
        grid=(NP // BLK,),
        in_specs=[
            pl.BlockSpec((BLK, D), lambda i: (i, 0)),
            pl.BlockSpec((D, D), lambda i: (0, 0)),
            pl.BlockSpec((BLK, 1), lambda i: (i, 0)),
            pl.BlockSpec((BLK, 1), lambda i: (i, 0)),
        ],
        out_specs=pl.BlockSpec((BLK, D), lambda i: (i, 0)),
        out_shape=jax.ShapeDtypeStruct((NP, D), jnp.float32),
    )(x_p, W1, k0, k1)


def _tc2_body(p0_ref, p1_ref, g1_ref, k0_ref, k1_ref, b1_ref, w2_ref, o_ref):
    d = _dcol(k0_ref, k1_ref)
    acc = p0_ref[...] + p1_ref[...] - g1_ref[...]
    h1 = jnp.maximum(d * acc + b1_ref[...], 0.0)
    o_ref[...] = (
        jnp.dot(h1, w2_ref[...], preferred_element_type=jnp.float32) * d
    )


def _tc2(p0, p1, g1, k0, k1, b1, W2):
    return pl.pallas_call(
        _tc2_body,
        grid=(NP // BLK,),
        in_specs=[
            pl.BlockSpec((BLK, D), lambda i: (i, 0)),
            pl.BlockSpec((BLK, D), lambda i: (i, 0)),
            pl.BlockSpec((BLK, D), lambda i: (i, 0)),
            pl.BlockSpec((BLK, 1), lambda i: (i, 0)),
            pl.BlockSpec((BLK, 1), lambda i: (i, 0)),
            pl.BlockSpec((1, D), lambda i: (0, 0)),
            pl.BlockSpec((D, D), lambda i: (0, 0)),
        ],
        out_specs=pl.BlockSpec((BLK, D), lambda i: (i, 0)),
        out_shape=jax.ShapeDtypeStruct((NP, D), jnp.float32),
    )(p0, p1, g1, k0, k1, b1, W2)


def _tc3_body(q0_ref, q1_ref, g2_ref, k0_ref, k1_ref, b2_ref, wm1_ref,
              bm1_ref, wm2_ref, bm2_ref, o_ref):
    d = _dcol(k0_ref, k1_ref)
    h2 = d * (q0_ref[...] + q1_ref[...] - g2_ref[...]) + b2_ref[...]
    h3 = jnp.maximum(
        jnp.dot(h2, wm1_ref[...], preferred_element_type=jnp.float32)
        + bm1_ref[...],
        0.0,
    )
    o_ref[...] = (
        jnp.dot(h3, wm2_ref[...], preferred_element_type=jnp.float32)
        + bm2_ref[...]
    )


def _tc3(q0, q1, g2, k0, k1, b2, Wm1, bm1, Wm2, bm2):
    return pl.pallas_call(
        _tc3_body,
        grid=(NP // BLK,),
        in_specs=[
            pl.BlockSpec((BLK, D), lambda i: (i, 0)),
            pl.BlockSpec((BLK, D), lambda i: (i, 0)),
            pl.BlockSpec((BLK, D), lambda i: (i, 0)),
            pl.BlockSpec((BLK, 1), lambda i: (i, 0)),
            pl.BlockSpec((BLK, 1), lambda i: (i, 0)),
            pl.BlockSpec((1, D), lambda i: (0, 0)),
            pl.BlockSpec((D, D), lambda i: (0, 0)),
            pl.BlockSpec((1, D), lambda i: (0, 0)),
            pl.BlockSpec((D, D), lambda i: (0, 0)),
            pl.BlockSpec((1, D), lambda i: (0, 0)),
        ],
        out_specs=pl.BlockSpec((BLK, D), lambda i: (i, 0)),
        out_shape=jax.ShapeDtypeStruct((NP, D), jnp.float32),
    )(q0, q1, g2, k0, k1, b2, Wm1, bm1, Wm2, bm2)


def kernel(x, edge_index, W1, b1, W2, b2, Wm1, bm1, Wm2, bm2):
    src = edge_index[0]
    dst = edge_index[1]
    pad = EPAD - EDGES
    pad_i = jnp.arange(pad, dtype=jnp.int32)
    src_p = jnp.concatenate([src, pad_i % N])
    dst_p = jnp.concatenate([dst, DUMMY + pad_i % (NP - N)])
    src_p = src_p.reshape(NW, NSEG, SEG, CHUNK)
    dst_p = dst_p.reshape(NW, NSEG, SEG, CHUNK)
    x_p = jnp.concatenate([x, jnp.zeros((NP - N, D), x.dtype)])

    deg0, deg1 = _deg_call(dst_p)
    k0 = deg0.reshape(NP, 1)
    k1 = deg1.reshape(NP, 1)
    g1 = _tc1(x_p, W1, k0, k1)
    p0, p1 = _scat_call(g1, src_p, dst_p)
    g2 = _tc2(p0, p1, g1, k0, k1, b1.reshape(1, D), W2)
    q0, q1 = _scat_call(g2, src_p, dst_p)
    out = _tc3(q0, q1, g2, k0, k1, b2.reshape(1, D), Wm1,
               bm1.reshape(1, D), Wm2, bm2.reshape(1, D))
    return out[:N]


# speedup vs baseline: 1.3328x; 1.1791x over previous
import jax
import jax.numpy as jnp
from jax import lax
from jax.experimental import pallas as pl
from jax.experimental.pallas import tpu as pltpu
from jax.experimental.pallas import tpu_sc as plsc

N = 10000
EDGES = 320000
D = 128
NP = 10240
NC = 2
NS = 16
NW = NC * NS
CHUNK = 96
NCHUNK = 108
EPT = NCHUNK * CHUNK
EPAD = EPT * NW
SEG = 36
NSEG = NCHUNK // SEG
NBUF = 3
RPT = NP // NS
BLK = 1024
DUMMY = N

_mesh = plsc.VectorSubcoreMesh(
    core_axis_name="c", subcore_axis_name="s", num_cores=NC, num_subcores=NS
)


def _tile_ids():
    cid = lax.axis_index("c")
    sid = lax.axis_index("s")
    return cid, sid, cid * NS + sid


def _deg_body(dst_hbm, out0_hbm, out1_hbm, dstv, ones_v, zbuf, accd,
              ssem0, ssem1):
    cid, sid, wid = _tile_ids()
    for i in range(RPT // 16):
        zbuf[pl.ds(i * 16, 16)] = jnp.zeros((16,), jnp.float32)
    for i in range(CHUNK // 16):
        ones_v[pl.ds(i * 16, 16)] = jnp.ones((16,), jnp.float32)
    row0 = pl.multiple_of(sid * RPT, 8)
    pltpu.sync_copy(zbuf, accd.at[pl.ds(row0, RPT)])
    plsc.subcore_barrier()

    sems = (ssem0, ssem1)

    def fire(j, b):
        pltpu.async_copy(ones_v, accd.at[dstv.at[j]], sems[b], add=True)

    def drain(j, b):
        pltpu.make_async_copy(ones_v, accd.at[dstv.at[j]], sems[b]).wait()

    for seg in range(NSEG):
        if seg > 0:
            drain(0, 0)
            drain(1, 1)
        pltpu.sync_copy(dst_hbm.at[wid, seg], dstv)
        fire(0, 0)
        fire(1, 1)

        def body(i, carry):
            j0 = 2 + 2 * i
            drain(j0, 0)
            fire(j0, 0)
            drain(j0 + 1, 1)
            fire(j0 + 1, 1)
            return carry

        lax.fori_loop(0, (SEG - 2) // 2, body, 0)
    drain(0, 0)
    drain(1, 1)
    plsc.subcore_barrier()

    @pl.when(cid == 0)
    def _():
        pltpu.sync_copy(accd.at[pl.ds(row0, RPT)], out0_hbm.at[pl.ds(row0, RPT)])

    @pl.when(cid == 1)
    def _():
        pltpu.sync_copy(accd.at[pl.ds(row0, RPT)], out1_hbm.at[pl.ds(row0, RPT)])


_deg_call = pl.kernel(
    _deg_body,
    out_type=(
        jax.ShapeDtypeStruct((NP,), jnp.float32),
        jax.ShapeDtypeStruct((NP,), jnp.float32),
    ),
    mesh=_mesh,
    scratch_types=[
        pltpu.VMEM((SEG, CHUNK), jnp.int32),
        pltpu.VMEM((CHUNK,), jnp.float32),
        pltpu.VMEM((RPT,), jnp.float32),
        pltpu.VMEM_SHARED((NP,), jnp.float32),
        pltpu.SemaphoreType.DMA,
        pltpu.SemaphoreType.DMA,
    ],
)


def _scat_body(g_hbm, src_hbm, dst_hbm, out0_hbm, out1_hbm, srcv, dstv,
               rows, acc, gsem0, gsem1, gsem2, ssem0, ssem1, ssem2):
    cid, sid, wid = _tile_ids()
    row0 = pl.multiple_of(sid * RPT, 8)
    pltpu.sync_copy(g_hbm.at[pl.ds(row0, RPT)], acc.at[pl.ds(row0, RPT)])
    plsc.subcore_barrier()

    gsems = (gsem0, gsem1, gsem2)
    sems = (ssem0, ssem1, ssem2)

    def fire_gather(j, b):
        pltpu.async_copy(g_hbm.at[srcv.at[j]], rows.at[b], gsems[b])

    def wait_gather(j, b):
        pltpu.make_async_copy(g_hbm.at[srcv.at[j]], rows.at[b], gsems[b]).wait()

    def fire_scatter(j, b):
        pltpu.async_copy(rows.at[b], acc.at[dstv.at[j]], sems[b], add=True)

    def drain_scatter(j, b):
        pltpu.make_async_copy(rows.at[b], acc.at[dstv.at[j]], sems[b]).wait()

    for seg in range(NSEG):
        pltpu.sync_copy(src_hbm.at[wid, seg], srcv)
        pltpu.sync_copy(dst_hbm.at[wid, seg], dstv)
        fire_gather(0, 0)
        fire_gather(1, 1)
        wait_gather(0, 0)
        fire_scatter(0, 0)
        fire_gather(2, 2)
        wait_gather(1, 1)
        fire_scatter(1, 1)

        def body(m, carry):
            j0 = 3 + 3 * m
            for t in range(NBUF):
                j = j0 + t
                drain_scatter(j - 3, t)
                fire_gather(j, t)
                wait_gather(j - 1, (t + 2) % NBUF)
                fire_scatter(j - 1, (t + 2) % NBUF)
            return carry

        lax.fori_loop(0, (SEG - 3) // 3, body, 0)
        wait_gather(SEG - 1, 2)
        fire_scatter(SEG - 1, 2)
        drain_scatter(SEG - 3, 0)
        drain_scatter(SEG - 2, 1)
        drain_scatter(SEG - 1, 2)
    plsc.subcore_barrier()

    @pl.when(cid == 0)
    def _():
        pltpu.sync_copy(acc.at[pl.ds(row0, RPT)], out0_hbm.at[pl.ds(row0, RPT)])

    @pl.when(cid == 1)
    def _():
        pltpu.sync_copy(acc.at[pl.ds(row0, RPT)], out1_hbm.at[pl.ds(row0, RPT)])


_scat_call = pl.kernel(
    _scat_body,
    out_type=(
        jax.ShapeDtypeStruct((NP, D), jnp.float32),
        jax.ShapeDtypeStruct((NP, D), jnp.float32),
    ),
    mesh=_mesh,
    scratch_types=[
        pltpu.VMEM((SEG, CHUNK), jnp.int32),
        pltpu.VMEM((SEG, CHUNK), jnp.int32),
        pltpu.VMEM((NBUF, CHUNK, D), jnp.float32),
        pltpu.VMEM_SHARED((NP, D), jnp.float32),
        pltpu.SemaphoreType.DMA,
        pltpu.SemaphoreType.DMA,
        pltpu.SemaphoreType.DMA,
        pltpu.SemaphoreType.DMA,
        pltpu.SemaphoreType.DMA,
        pltpu.SemaphoreType.DMA,
    ],
)


def _dcol(k0_ref, k1_ref):
    return lax.rsqrt(k0_ref[...] + k1_ref[...] + 1.0)


def _tc1_body(x_ref, w_ref, k0_ref, k1_ref, o_ref):
    o_ref[...] = (
        jnp.dot(x_ref[...], w_ref[...], preferred_element_type=jnp.float32)
        * _dcol(k0_ref, k1_ref)
    )


def _tc1(x_p, W1, k0, k1):
    return pl.pallas_call(
        _tc1_body,
        grid=(NP // BLK,),
        in_specs=[
            pl.BlockSpec((BLK, D), lambda i: (i, 0)),
            pl.BlockSpec((D, D), lambda i: (0, 0)),
            pl.BlockSpec((BLK, 1), lambda i: (i, 0)),
            pl.BlockSpec((BLK, 1), lambda i: (i, 0)),
        ],
        out_specs=pl.BlockSpec((BLK, D), lambda i: (i, 0)),
        out_shape=jax.ShapeDtypeStruct((NP, D), jnp.float32),
    )(x_p, W1, k0, k1)


def _tc2_body(p0_ref, p1_ref, g1_ref, k0_ref, k1_ref, b1_ref, w2_ref, o_ref):
    d = _dcol(k0_ref, k1_ref)
    acc = p0_ref[...] + p1_ref[...] - g1_ref[...]
    h1 = jnp.maximum(d * acc + b1_ref[...], 0.0)
    o_ref[...] = (
        jnp.dot(h1, w2_ref[...], preferred_element_type=jnp.float32) * d
    )


def _tc2(p0, p1, g1, k0, k1, b1, W2):
    return pl.pallas_call(
        _tc2_body,
        grid=(NP // BLK,),
        in_specs=[
            pl.BlockSpec((BLK, D), lambda i: (i, 0)),
            pl.BlockSpec((BLK, D), lambda i: (i, 0)),
            pl.BlockSpec((BLK, D), lambda i: (i, 0)),
            pl.BlockSpec((BLK, 1), lambda i: (i, 0)),
            pl.BlockSpec((BLK, 1), lambda i: (i, 0)),
            pl.BlockSpec((1, D), lambda i: (0, 0)),
            pl.BlockSpec((D, D), lambda i: (0, 0)),
        ],
        out_specs=pl.BlockSpec((BLK, D), lambda i: (i, 0)),
        out_shape=jax.ShapeDtypeStruct((NP, D), jnp.float32),
    )(p0, p1, g1, k0, k1, b1, W2)


def _tc3_body(q0_ref, q1_ref, g2_ref, k0_ref, k1_ref, b2_ref, wm1_ref,
              bm1_ref, wm2_ref, bm2_ref, o_ref):
    d = _dcol(k0_ref, k1_ref)
    h2 = d * (q0_ref[...] + q1_ref[...] - g2_ref[...]) + b2_ref[...]
    h3 = jnp.maximum(
        jnp.dot(h2, wm1_ref[...], preferred_element_type=jnp.float32)
        + bm1_ref[...],
        0.0,
    )
    o_ref[...] = (
        jnp.dot(h3, wm2_ref[...], preferred_element_type=jnp.float32)
        + bm2_ref[...]
    )


def _tc3(q0, q1, g2, k0, k1, b2, Wm1, bm1, Wm2, bm2):
    return pl.pallas_call(
        _tc3_body,
        grid=(NP // BLK,),
        in_specs=[
            pl.BlockSpec((BLK, D), lambda i: (i, 0)),
            pl.BlockSpec((BLK, D), lambda i: (i, 0)),
            pl.BlockSpec((BLK, D), lambda i: (i, 0)),
            pl.BlockSpec((BLK, 1), lambda i: (i, 0)),
            pl.BlockSpec((BLK, 1), lambda i: (i, 0)),
            pl.BlockSpec((1, D), lambda i: (0, 0)),
            pl.BlockSpec((D, D), lambda i: (0, 0)),
            pl.BlockSpec((1, D), lambda i: (0, 0)),
            pl.BlockSpec((D, D), lambda i: (0, 0)),
            pl.BlockSpec((1, D), lambda i: (0, 0)),
        ],
        out_specs=pl.BlockSpec((BLK, D), lambda i: (i, 0)),
        out_shape=jax.ShapeDtypeStruct((N, D), jnp.float32),
    )(q0, q1, g2, k0, k1, b2, Wm1, bm1, Wm2, bm2)


def kernel(x, edge_index, W1, b1, W2, b2, Wm1, bm1, Wm2, bm2):
    src = edge_index[0]
    dst = edge_index[1]
    pad = EPAD - EDGES
    pad_i = jnp.arange(pad, dtype=jnp.int32)
    src_p = jnp.concatenate([src, pad_i & 8191])
    dst_p = jnp.concatenate([dst, DUMMY + (pad_i & 127)])
    src_p = src_p.reshape(NW, NSEG, SEG, CHUNK)
    dst_p = dst_p.reshape(NW, NSEG, SEG, CHUNK)

    deg0, deg1 = _deg_call(dst_p)
    k0 = deg0.reshape(NP, 1)
    k1 = deg1.reshape(NP, 1)
    g1 = _tc1(x, W1, k0, k1)
    p0, p1 = _scat_call(g1, src_p, dst_p)
    g2 = _tc2(p0, p1, g1, k0, k1, b1.reshape(1, D), W2)
    q0, q1 = _scat_call(g2, src_p, dst_p)
    return _tc3(q0, q1, g2, k0, k1, b2.reshape(1, D), Wm1,
                bm1.reshape(1, D), Wm2, bm2.reshape(1, D))
